# Initial kernel scaffold; baseline (speedup 1.0000x reference)
import jax
import jax.numpy as jnp
from jax.experimental import pallas as pl


def kernel(x, table):
    raise NotImplementedError("write your pallas kernel here")


# speedup vs baseline: 1.0228x; 1.0228x over previous
import functools

import jax
import jax.numpy as jnp
from jax import lax
from jax.experimental import pallas as pl
from jax.experimental.pallas import tpu as pltpu
from jax.experimental.pallas import tpu_sc as plsc

_NC = 2
_NS = 16
_NW = _NC * _NS
_CH = 128


@functools.lru_cache(maxsize=None)
def _make(V, D, B):
    assert B % (_NW * _CH) == 0
    nchunks = B // (_NW * _CH)
    per_w = nchunks * _CH
    mesh = plsc.VectorSubcoreMesh(core_axis_name="c", subcore_axis_name="s")

    @functools.partial(
        pl.kernel,
        mesh=mesh,
        out_type=jax.ShapeDtypeStruct((B, D), jnp.float32),
        scratch_types=[
            pltpu.VMEM((nchunks, _CH), jnp.int32),
            pltpu.VMEM((_CH, D), jnp.float32),
            pltpu.SemaphoreType.DMA,
        ],
        compiler_params=pltpu.CompilerParams(use_tc_tiling_on_sc=False),
    )
    def k(table_hbm, idx_hbm, out_hbm, idx_v, rows_v, sem):
        wid = lax.axis_index("s") * _NC + lax.axis_index("c")
        pltpu.sync_copy(idx_hbm.at[wid], idx_v)
        base = wid * per_w

        @pl.loop(0, nchunks)
        def _(j):
            pltpu.async_copy(table_hbm.at[idx_v.at[j]], rows_v, sem).wait()
            pltpu.sync_copy(rows_v, out_hbm.at[pl.ds(base + j * _CH, _CH)])

    return k


def kernel(x, table):
    Bx, H = x.shape
    V, D = table.shape
    idx = x.astype(jnp.int32).reshape(_NW, -1, _CH)
    out = _make(V, D, Bx * H)(table, idx)
    return out.reshape(Bx, H, D)


# speedup vs baseline: 1.1139x; 1.0891x over previous
import functools

import jax
import jax.numpy as jnp
from jax import lax
from jax.experimental import pallas as pl
from jax.experimental.pallas import tpu as pltpu
from jax.experimental.pallas import tpu_sc as plsc

_NC = 2
_NS = 16
_NW = _NC * _NS
_CH = 128
_G = 10


@functools.lru_cache(maxsize=None)
def _make(V, D, B):
    assert B % (_NW * _CH) == 0
    nchunks = B // (_NW * _CH)
    assert nchunks % (2 * _G) == 0
    ngroups = nchunks // _G
    per_w = nchunks * _CH
    grows = _G * _CH
    mesh = plsc.VectorSubcoreMesh(core_axis_name="c", subcore_axis_name="s")

    @functools.partial(
        pl.kernel,
        mesh=mesh,
        out_type=jax.ShapeDtypeStruct((B, D), jnp.float32),
        scratch_types=[
            pltpu.VMEM((nchunks, _CH), jnp.int32),
            pltpu.VMEM((2, grows, D), jnp.float32),
            pltpu.SemaphoreType.DMA,
            pltpu.SemaphoreType.DMA,
            pltpu.SemaphoreType.DMA,
            pltpu.SemaphoreType.DMA,
        ],
        compiler_params=pltpu.CompilerParams(use_tc_tiling_on_sc=False),
    )
    def k(table_hbm, idx_hbm, out_hbm, idx_v, rows_v, g0, g1, s0, s1):
        gsem = (g0, g1)
        ssem = (s0, s1)
        wid = lax.axis_index("s") * _NC + lax.axis_index("c")
        pltpu.sync_copy(idx_hbm.at[wid], idx_v)
        base = wid * per_w

        def fire(g, p):
            for c in range(_G):
                pltpu.async_copy(
                    table_hbm.at[idx_v.at[g * _G + c]],
                    rows_v.at[p, pl.ds(c * _CH, _CH)],
                    gsem[p],
                )

        def drain_gathers(p):
            pltpu.make_async_copy(
                table_hbm.at[pl.ds(0, grows)], rows_v.at[p], gsem[p]
            ).wait()

        def drain_store(p):
            pltpu.make_async_copy(
                rows_v.at[p], out_hbm.at[pl.ds(base, grows)], ssem[p]
            ).wait()

        fire(0, 0)

        @pl.loop(0, ngroups, step=2)
        def _(i0):
            for p in (0, 1):
                g = i0 + p

                @pl.when(g + 1 < ngroups)
                def _():
                    @pl.when(g >= 1)
                    def _():
                        drain_store(1 - p)

                    fire(g + 1, 1 - p)

                drain_gathers(p)
                pltpu.async_copy(
                    rows_v.at[p],
                    out_hbm.at[pl.ds(base + g * grows, grows)],
                    ssem[p],
                )

        drain_store(0)
        drain_store(1)

    return k


def kernel(x, table):
    Bx, H = x.shape
    V, D = table.shape
    idx = x.astype(jnp.int32).reshape(_NW, -1, _CH)
    out = _make(V, D, Bx * H)(table, idx)
    return out.reshape(Bx, H, D)


# speedup vs baseline: 1.6360x; 1.4688x over previous
import functools

import jax
import jax.numpy as jnp
from jax import lax
from jax.experimental import pallas as pl
from jax.experimental.pallas import tpu as pltpu
from jax.experimental.pallas import tpu_sc as plsc

_NC = 2
_NS = 16
_NW = _NC * _NS
_L = 16


@functools.lru_cache(maxsize=None)
def _make(V, D, H, B):
    assert D == 32 and V % 4 == 0
    bpw = B // _NW
    nbb = bpw // 128
    nslab = H * nbb
    h_lo = (H // 8) * 8
    mesh = plsc.VectorSubcoreMesh(core_axis_name="c", subcore_axis_name="s")

    @functools.partial(
        pl.kernel,
        mesh=mesh,
        out_type=jax.ShapeDtypeStruct((H, D, B), jnp.float32),
        scratch_types=[
            pltpu.VMEM((H, bpw), jnp.int32),
            pltpu.VMEM((2, 128), jnp.int32),
            pltpu.VMEM((2, 128), jnp.int32),
            pltpu.VMEM((2, 128, 128), jnp.float32),
            pltpu.VMEM((2, D, 128), jnp.float32),
            pltpu.SemaphoreType.DMA,
            pltpu.SemaphoreType.DMA,
            pltpu.SemaphoreType.DMA,
            pltpu.SemaphoreType.DMA,
        ],
        compiler_params=pltpu.CompilerParams(
            use_tc_tiling_on_sc=True, needs_layout_passes=False),
    )
    def k(tbl_hbm, xt_hbm, out_hbm, idx_all, idx_q, sub_q, gbuf, ostage,
          g0, g1, s0, s1):
        gsem = (g0, g1)
        ssem = (s0, s1)
        wid = lax.axis_index("s") * _NC + lax.axis_index("c")
        col0 = wid * bpw
        pltpu.sync_copy(xt_hbm.at[pl.ds(0, h_lo), pl.ds(col0, bpw)],
                        idx_all.at[pl.ds(0, h_lo)])
        pltpu.sync_copy(xt_hbm.at[pl.ds(h_lo, H - h_lo), pl.ds(col0, bpw)],
                        idx_all.at[pl.ds(h_lo, H - h_lo)])

        def prep(s, p):
            h = s % H
            bbl = s // H
            for kk in range(8):
                v = idx_all[h, pl.ds(bbl * 128 + kk * _L, _L)]
                idx_q[p, pl.ds(kk * _L, _L)] = lax.shift_right_logical(v, 2)
                sub_q[p, pl.ds(kk * _L, _L)] = lax.bitwise_and(v, 3)

        def fire_gather(p):
            pltpu.async_copy(tbl_hbm.at[idx_q.at[p]], gbuf.at[p], gsem[p])

        rowv = [jnp.arange(_L, dtype=jnp.int32) + kk * _L for kk in range(8)]

        prep(0, 0)
        fire_gather(0)

        @pl.loop(0, nslab, step=2)
        def _(s0i):
            for p in (0, 1):
                s = s0i + p

                @pl.when(s + 1 < nslab)
                def _():
                    prep(s + 1, 1 - p)
                    fire_gather(1 - p)

                pltpu.make_async_copy(
                    tbl_hbm.at[pl.ds(0, 128)], gbuf.at[p], gsem[p]).wait()

                @pl.when(s >= 2)
                def _():
                    pltpu.make_async_copy(
                        ostage.at[p],
                        out_hbm.at[0, pl.ds(0, D), pl.ds(0, 128)],
                        ssem[p]).wait()

                colb = [sub_q[p, pl.ds(kk * _L, _L)] * 32 for kk in range(8)]

                @pl.loop(0, D)
                def _(d):
                    for kk in range(8):
                        val = plsc.load_gather(
                            gbuf.at[p], [rowv[kk], colb[kk] + d])
                        ostage[p, d, pl.ds(kk * _L, _L)] = val

                h = s % H
                bbl = s // H
                pltpu.async_copy(
                    ostage.at[p],
                    out_hbm.at[h, pl.ds(0, D), pl.ds(col0 + bbl * 128, 128)],
                    ssem[p])

        for p in (0, 1):
            pltpu.make_async_copy(
                ostage.at[p],
                out_hbm.at[0, pl.ds(0, D), pl.ds(0, 128)],
                ssem[p]).wait()

    return k


def kernel(x, table):
    Bx, H = x.shape
    V, D = table.shape
    xt = x.astype(jnp.int32).T
    tbl128 = table.reshape(V // 4, 4 * D)
    out3 = _make(V, D, H, Bx)(tbl128, xt)
    return out3.transpose(2, 0, 1)
